# initial kernel scaffold (unmeasured)
import jax
import jax.numpy as jnp
from jax import lax
from jax.experimental import pallas as pl
from jax.experimental.pallas import tpu as pltpu

T_LOCAL = 4096
D = 2048
F = 4096
E_LOCAL = 4
TILE_T = 512


def _gather_body(x_ref, a_ref, xall_ref, aall_ref, sems):
    my_x = lax.axis_index("x")
    my_y = lax.axis_index("y")
    peer = (my_x, 1 - my_y)

    barrier = pltpu.get_barrier_semaphore()
    pl.semaphore_signal(
        barrier, inc=1, device_id=peer, device_id_type=pl.DeviceIdType.MESH
    )
    pl.semaphore_wait(barrier, 1)

    xall_ref[pl.ds(my_y * T_LOCAL, T_LOCAL), :] = x_ref[...]
    aall_ref[pl.ds(my_y, 1), :] = a_ref[...]

    rdma_x = pltpu.make_async_remote_copy(
        src_ref=x_ref,
        dst_ref=xall_ref.at[pl.ds(my_y * T_LOCAL, T_LOCAL), :],
        send_sem=sems.at[0],
        recv_sem=sems.at[1],
        device_id=peer,
        device_id_type=pl.DeviceIdType.MESH,
    )
    rdma_a = pltpu.make_async_remote_copy(
        src_ref=a_ref,
        dst_ref=aall_ref.at[pl.ds(my_y, 1), :],
        send_sem=sems.at[2],
        recv_sem=sems.at[3],
        device_id=peer,
        device_id_type=pl.DeviceIdType.MESH,
    )
    rdma_x.start()
    rdma_a.start()
    rdma_x.wait()
    rdma_a.wait()


def _moe_body(a_ref, x_ref, w1_ref, w2_ref, out_ref):
    e = pl.program_id(1)
    x = x_ref[...]
    h = jnp.maximum(
        jnp.dot(x, w1_ref[0], preferred_element_type=jnp.float32), 0.0
    ).astype(jnp.bfloat16)
    o = jnp.dot(h, w2_ref[0], preferred_element_type=jnp.float32)
    mask = a_ref[...] == e
    contrib = jnp.where(mask, o, 0.0).astype(jnp.bfloat16)

    @pl.when(e == 0)
    def _():
        out_ref[...] = contrib

    @pl.when(e != 0)
    def _():
        out_ref[...] += contrib


def _reduce_body(p_ref, out_ref, comm_ref, sems):
    my_x = lax.axis_index("x")
    my_y = lax.axis_index("y")
    peer = (my_x, 1 - my_y)

    barrier = pltpu.get_barrier_semaphore()
    pl.semaphore_signal(
        barrier, inc=1, device_id=peer, device_id_type=pl.DeviceIdType.MESH
    )
    pl.semaphore_wait(barrier, 1)

    rdma = pltpu.make_async_remote_copy(
        src_ref=p_ref.at[pl.ds((1 - my_y) * T_LOCAL, T_LOCAL), :],
        dst_ref=comm_ref,
        send_sem=sems.at[0],
        recv_sem=sems.at[1],
        device_id=peer,
        device_id_type=pl.DeviceIdType.MESH,
    )
    rdma.start()
    rdma.wait()

    out_ref[...] = (
        p_ref[pl.ds(my_y * T_LOCAL, T_LOCAL), :].astype(jnp.float32)
        + comm_ref[...].astype(jnp.float32)
    )


def kernel(x, assign, W1, W2):
    xb = x.astype(jnp.bfloat16)
    w1b = W1.astype(jnp.bfloat16)
    w2b = W2.astype(jnp.bfloat16)
    a2 = assign.reshape(1, T_LOCAL)

    xall, aall = pl.pallas_call(
        _gather_body,
        out_shape=(
            jax.ShapeDtypeStruct((2 * T_LOCAL, D), jnp.bfloat16),
            jax.ShapeDtypeStruct((2, T_LOCAL), jnp.int32),
        ),
        in_specs=[
            pl.BlockSpec(memory_space=pltpu.VMEM),
            pl.BlockSpec(memory_space=pltpu.VMEM),
        ],
        out_specs=(
            pl.BlockSpec(memory_space=pltpu.VMEM),
            pl.BlockSpec(memory_space=pltpu.VMEM),
        ),
        scratch_shapes=[pltpu.SemaphoreType.DMA((4,))],
        compiler_params=pltpu.CompilerParams(collective_id=0),
    )(xb, a2)

    my_y = lax.axis_index("y")
    a_local = aall.reshape(2 * T_LOCAL, 1) - 4 * my_y

    n_t = 2 * T_LOCAL // TILE_T
    partial = pl.pallas_call(
        _moe_body,
        grid=(n_t, E_LOCAL),
        in_specs=[
            pl.BlockSpec((TILE_T, 1), lambda t, e: (t, 0)),
            pl.BlockSpec((TILE_T, D), lambda t, e: (t, 0)),
            pl.BlockSpec((1, D, F), lambda t, e: (e, 0, 0)),
            pl.BlockSpec((1, F, D), lambda t, e: (e, 0, 0)),
        ],
        out_specs=pl.BlockSpec((TILE_T, D), lambda t, e: (t, 0)),
        out_shape=jax.ShapeDtypeStruct((2 * T_LOCAL, D), jnp.bfloat16),
        compiler_params=pltpu.CompilerParams(
            dimension_semantics=("arbitrary", "arbitrary"),
        ),
    )(a_local, xall, w1b, w2b)

    out = pl.pallas_call(
        _reduce_body,
        out_shape=jax.ShapeDtypeStruct((T_LOCAL, D), jnp.float32),
        in_specs=[pl.BlockSpec(memory_space=pltpu.VMEM)],
        out_specs=pl.BlockSpec(memory_space=pltpu.VMEM),
        scratch_shapes=[
            pltpu.VMEM((T_LOCAL, D), jnp.bfloat16),
            pltpu.SemaphoreType.DMA((2,)),
        ],
        compiler_params=pltpu.CompilerParams(collective_id=1),
    )(partial)
    return out


# baseline (device time: 1804444 ns/iter reference)
import jax
import jax.numpy as jnp
from jax import lax
from jax.experimental import pallas as pl
from jax.experimental.pallas import tpu as pltpu

T_LOCAL = 4096
D = 2048
F = 4096
E_LOCAL = 4
TILE_T = 512
TILE_F = 1024


def _gather_body(x_ref, a_ref, xall_ref, aall_ref, sems):
    my_x = lax.axis_index("x")
    my_y = lax.axis_index("y")
    peer = (my_x, 1 - my_y)

    barrier = pltpu.get_barrier_semaphore()
    pl.semaphore_signal(
        barrier, inc=1, device_id=peer, device_id_type=pl.DeviceIdType.MESH
    )
    pl.semaphore_wait(barrier, 1)

    xall_ref[pl.ds(my_y * T_LOCAL, T_LOCAL), :] = x_ref[...]
    aall_ref[pl.ds(my_y, 1), :] = a_ref[...]

    rdma_x = pltpu.make_async_remote_copy(
        src_ref=x_ref,
        dst_ref=xall_ref.at[pl.ds(my_y * T_LOCAL, T_LOCAL), :],
        send_sem=sems.at[0],
        recv_sem=sems.at[1],
        device_id=peer,
        device_id_type=pl.DeviceIdType.MESH,
    )
    rdma_a = pltpu.make_async_remote_copy(
        src_ref=a_ref,
        dst_ref=aall_ref.at[pl.ds(my_y, 1), :],
        send_sem=sems.at[2],
        recv_sem=sems.at[3],
        device_id=peer,
        device_id_type=pl.DeviceIdType.MESH,
    )
    rdma_x.start()
    rdma_a.start()
    rdma_x.wait()
    rdma_a.wait()


def _moe_body(a_ref, x_ref, w1_ref, w2_ref, out_ref):
    e = pl.program_id(1)
    f = pl.program_id(2)
    x = x_ref[...]
    h = jnp.maximum(
        jnp.dot(x, w1_ref[0], preferred_element_type=jnp.float32), 0.0
    ).astype(jnp.bfloat16)
    o = jnp.dot(h, w2_ref[0], preferred_element_type=jnp.float32)
    mask = a_ref[...] == e
    contrib = jnp.where(mask, o, 0.0).astype(jnp.bfloat16)

    @pl.when((e == 0) & (f == 0))
    def _():
        out_ref[...] = contrib

    @pl.when((e != 0) | (f != 0))
    def _():
        out_ref[...] += contrib


def _reduce_body(p_ref, out_ref, local_ref, comm_ref, sems):
    my_x = lax.axis_index("x")
    my_y = lax.axis_index("y")
    peer = (my_x, 1 - my_y)

    barrier = pltpu.get_barrier_semaphore()
    pl.semaphore_signal(
        barrier, inc=1, device_id=peer, device_id_type=pl.DeviceIdType.MESH
    )
    pl.semaphore_wait(barrier, 1)

    rdma = pltpu.make_async_remote_copy(
        src_ref=p_ref.at[pl.ds((1 - my_y) * T_LOCAL, T_LOCAL), :],
        dst_ref=comm_ref,
        send_sem=sems.at[0],
        recv_sem=sems.at[1],
        device_id=peer,
        device_id_type=pl.DeviceIdType.MESH,
    )
    rdma.start()

    local_copy = pltpu.make_async_copy(
        p_ref.at[pl.ds(my_y * T_LOCAL, T_LOCAL), :], local_ref, sems.at[2]
    )
    local_copy.start()
    local_copy.wait()
    rdma.wait()

    out_ref[...] = local_ref[...] + comm_ref[...]


def kernel(x, assign, W1, W2):
    xb = x.astype(jnp.bfloat16)
    w1b = W1.astype(jnp.bfloat16)
    w2b = W2.astype(jnp.bfloat16)
    a2 = assign.reshape(1, T_LOCAL)

    xall, aall = pl.pallas_call(
        _gather_body,
        out_shape=(
            jax.ShapeDtypeStruct((2 * T_LOCAL, D), jnp.bfloat16),
            jax.ShapeDtypeStruct((2, T_LOCAL), jnp.int32),
        ),
        in_specs=[
            pl.BlockSpec(memory_space=pltpu.VMEM),
            pl.BlockSpec(memory_space=pltpu.VMEM),
        ],
        out_specs=(
            pl.BlockSpec(memory_space=pltpu.VMEM),
            pl.BlockSpec(memory_space=pltpu.VMEM),
        ),
        scratch_shapes=[pltpu.SemaphoreType.DMA((4,))],
        compiler_params=pltpu.CompilerParams(collective_id=0),
    )(xb, a2)

    my_y = lax.axis_index("y")
    a_local = aall.reshape(2 * T_LOCAL, 1) - 4 * my_y

    n_t = 2 * T_LOCAL // TILE_T
    n_f = F // TILE_F
    partial = pl.pallas_call(
        _moe_body,
        grid=(n_t, E_LOCAL, n_f),
        in_specs=[
            pl.BlockSpec((TILE_T, 1), lambda t, e, f: (t, 0)),
            pl.BlockSpec((TILE_T, D), lambda t, e, f: (t, 0)),
            pl.BlockSpec((1, D, TILE_F), lambda t, e, f: (e, 0, f)),
            pl.BlockSpec((1, TILE_F, D), lambda t, e, f: (e, f, 0)),
        ],
        out_specs=pl.BlockSpec((TILE_T, D), lambda t, e, f: (t, 0)),
        out_shape=jax.ShapeDtypeStruct((2 * T_LOCAL, D), jnp.bfloat16),
        compiler_params=pltpu.CompilerParams(
            dimension_semantics=("arbitrary", "arbitrary", "arbitrary"),
        ),
    )(a_local, xall, w1b, w2b)

    out = pl.pallas_call(
        _reduce_body,
        out_shape=jax.ShapeDtypeStruct((T_LOCAL, D), jnp.bfloat16),
        in_specs=[pl.BlockSpec(memory_space=pl.ANY)],
        out_specs=pl.BlockSpec(memory_space=pltpu.VMEM),
        scratch_shapes=[
            pltpu.VMEM((T_LOCAL, D), jnp.bfloat16),
            pltpu.VMEM((T_LOCAL, D), jnp.bfloat16),
            pltpu.SemaphoreType.DMA((3,)),
        ],
        compiler_params=pltpu.CompilerParams(collective_id=1),
    )(partial)
    return out.astype(jnp.float32)
